# baseline (speedup 1.0000x reference)
import functools

import jax
import jax.numpy as jnp
from jax import lax
from jax.experimental import pallas as pl
from jax.experimental.pallas import tpu as pltpu
from jax.experimental.pallas import tpu_sc as plsc

L = 16
NC = 2
NS = 16
NW = NC * NS
CHUNK = 128
SUPER = 8192


def _make_sc_kernel(n_user, n_movie, dim, e):
    assert dim % L == 0 and e % (NW * SUPER) == 0
    epw = e // NW
    nch = SUPER // CHUNK
    nsuper = epw // SUPER
    mesh = plsc.VectorSubcoreMesh(core_axis_name="c", subcore_axis_name="s")

    @functools.partial(
        pl.kernel,
        mesh=mesh,
        compiler_params=pltpu.CompilerParams(
            needs_layout_passes=False, use_tc_tiling_on_sc=False),
        out_type=jax.ShapeDtypeStruct((e,), jnp.float32),
        scratch_types=[
            pltpu.VMEM((SUPER,), jnp.int32),
            pltpu.VMEM((SUPER,), jnp.int32),
            pltpu.VMEM((CHUNK, dim), jnp.float32),
            pltpu.VMEM((CHUNK, dim), jnp.float32),
            pltpu.VMEM((CHUNK, dim), jnp.float32),
            pltpu.VMEM((CHUNK, dim), jnp.float32),
            pltpu.VMEM((epw,), jnp.float32),
            pltpu.SemaphoreType.DMA,
            pltpu.SemaphoreType.DMA,
            pltpu.SemaphoreType.DMA,
            pltpu.SemaphoreType.DMA,
        ],
    )
    def sc_kernel(xu, xm, eli, out, uidx_v, midx_v, ru0, ru1, rm0, rm1,
                  out_v, su0, su1, sm0, sm1):
        cid = lax.axis_index("c")
        sid = lax.axis_index("s")
        wid = sid * NC + cid
        base = wid * epw
        iota16 = lax.iota(jnp.int32, L)
        rbufs = ((ru0, rm0, su0, sm0), (ru1, rm1, su1, sm1))

        def fire(b, j):
            ru, rm, su, sm = rbufs[b]
            jj = pl.multiple_of(j * CHUNK, CHUNK)
            pltpu.async_copy(xu.at[uidx_v.at[pl.ds(jj, CHUNK)]], ru, su)
            pltpu.async_copy(xm.at[midx_v.at[pl.ds(jj, CHUNK)]], rm, sm)

        def wait(b):
            ru, rm, su, sm = rbufs[b]
            pltpu.make_async_copy(xu.at[pl.ds(0, CHUNK)], ru, su).wait()
            pltpu.make_async_copy(xm.at[pl.ds(0, CHUNK)], rm, sm).wait()

        def compute_chunk(b, out_off):
            ru, rm, _, _ = rbufs[b]

            def group_body(g, carry):
                rows = g * L + iota16
                terms = []
                for k in range(dim):
                    col = jnp.full((L,), k, jnp.int32)
                    uvec = plsc.load_gather(ru, [rows, col])
                    mvec = plsc.load_gather(rm, [rows, col])
                    terms.append(uvec * mvec)
                while len(terms) > 1:
                    terms = [terms[i] + terms[i + 1]
                             for i in range(0, len(terms), 2)]
                pred = 1.0 / (1.0 + jnp.exp(-terms[0]))
                out_v[pl.ds(out_off + g * L, L)] = pred
                return carry
            lax.fori_loop(0, CHUNK // L, group_body, 0)

        def super_body(s, carry):
            soff = pl.multiple_of(base + s * SUPER, SUPER)
            pltpu.sync_copy(eli.at[pl.ds(soff, SUPER)], uidx_v)
            pltpu.sync_copy(eli.at[pl.ds(soff + e, SUPER)], midx_v)

            fire(0, 0)

            def pair_body(jp, c2):
                j0 = jp * 2
                fire(1, j0 + 1)
                wait(0)
                compute_chunk(0, s * SUPER + j0 * CHUNK)

                @pl.when(j0 + 2 < nch)
                def _prefetch():
                    fire(0, j0 + 2)

                wait(1)
                compute_chunk(1, s * SUPER + (j0 + 1) * CHUNK)
                return c2
            lax.fori_loop(0, nch // 2, pair_body, 0)
            return carry

        lax.fori_loop(0, nsuper, super_body, 0)
        pltpu.sync_copy(out_v, out.at[pl.ds(base, epw)])

    return sc_kernel


def kernel(x_user, x_movie, edge_label_index):
    n_user, dim = x_user.shape
    n_movie, _ = x_movie.shape
    e = edge_label_index.shape[1]
    eli = edge_label_index.astype(jnp.int32).reshape(-1)
    sc = _make_sc_kernel(n_user, n_movie, dim, e)
    return sc(x_user, x_movie, eli)


# speedup vs baseline: 2.0930x; 2.0930x over previous
import functools

import jax
import jax.numpy as jnp
from jax import lax
from jax.experimental import pallas as pl
from jax.experimental.pallas import tpu as pltpu
from jax.experimental.pallas import tpu_sc as plsc

L = 16
NC = 2
NS = 16
NW = NC * NS
CHUNK = 128
SUPER = 8192


def _make_sc_kernel(n_user, n_movie, dim, e):
    assert dim % L == 0 and e % (NW * SUPER) == 0
    epw = e // NW
    nch = SUPER // CHUNK
    nsuper = epw // SUPER
    mesh = plsc.VectorSubcoreMesh(core_axis_name="c", subcore_axis_name="s")

    @functools.partial(
        pl.kernel,
        mesh=mesh,
        compiler_params=pltpu.CompilerParams(
            needs_layout_passes=False, use_tc_tiling_on_sc=False),
        out_type=jax.ShapeDtypeStruct((e,), jnp.float32),
        scratch_types=[
            pltpu.VMEM((SUPER,), jnp.int32),
            pltpu.VMEM((SUPER,), jnp.int32),
            pltpu.VMEM((CHUNK, dim), jnp.float32),
            pltpu.VMEM((CHUNK, dim), jnp.float32),
            pltpu.VMEM((CHUNK, dim), jnp.float32),
            pltpu.VMEM((CHUNK, dim), jnp.float32),
            pltpu.VMEM((epw,), jnp.float32),
            pltpu.VMEM((L * 17,), jnp.float32),
            pltpu.SemaphoreType.DMA,
            pltpu.SemaphoreType.DMA,
            pltpu.SemaphoreType.DMA,
            pltpu.SemaphoreType.DMA,
        ],
    )
    def sc_kernel(xu, xm, eli, out, uidx_v, midx_v, ru0, ru1, rm0, rm1,
                  out_v, tb, su0, su1, sm0, sm1):
        cid = lax.axis_index("c")
        sid = lax.axis_index("s")
        wid = sid * NC + cid
        base = wid * epw
        iota16 = lax.iota(jnp.int32, L)
        rbufs = ((ru0, rm0, su0, sm0), (ru1, rm1, su1, sm1))

        def fire(b, j):
            ru, rm, su, sm = rbufs[b]
            jj = pl.multiple_of(j * CHUNK, CHUNK)
            pltpu.async_copy(xu.at[uidx_v.at[pl.ds(jj, CHUNK)]], ru, su)
            pltpu.async_copy(xm.at[midx_v.at[pl.ds(jj, CHUNK)]], rm, sm)

        def wait(b):
            ru, rm, su, sm = rbufs[b]
            pltpu.make_async_copy(xu.at[pl.ds(0, CHUNK)], ru, su).wait()
            pltpu.make_async_copy(xm.at[pl.ds(0, CHUNK)], rm, sm).wait()

        def compute_chunk(b, out_off):
            ru, rm, _, _ = rbufs[b]

            def group_body(g, carry):
                for el in range(L):
                    row = g * L + el
                    acc = ru[row, pl.ds(0, L)] * rm[row, pl.ds(0, L)]
                    for k in range(1, dim // L):
                        acc = acc + (ru[row, pl.ds(k * L, L)]
                                     * rm[row, pl.ds(k * L, L)])
                    tb[pl.ds(el * 17, L)] = acc
                cols = [plsc.load_gather(tb, [iota16 * 17 + j])
                        for j in range(L)]
                while len(cols) > 1:
                    cols = [cols[i] + cols[i + 1]
                            for i in range(0, len(cols), 2)]
                pred = 1.0 / (1.0 + jnp.exp(-cols[0]))
                out_v[pl.ds(out_off + g * L, L)] = pred
                return carry
            lax.fori_loop(0, CHUNK // L, group_body, 0)

        def super_body(s, carry):
            soff = pl.multiple_of(base + s * SUPER, SUPER)
            pltpu.sync_copy(eli.at[pl.ds(soff, SUPER)], uidx_v)
            pltpu.sync_copy(eli.at[pl.ds(soff + e, SUPER)], midx_v)

            fire(0, 0)

            def pair_body(jp, c2):
                j0 = jp * 2
                fire(1, j0 + 1)
                wait(0)
                compute_chunk(0, s * SUPER + j0 * CHUNK)

                @pl.when(j0 + 2 < nch)
                def _prefetch():
                    fire(0, j0 + 2)

                wait(1)
                compute_chunk(1, s * SUPER + (j0 + 1) * CHUNK)
                return c2
            lax.fori_loop(0, nch // 2, pair_body, 0)
            return carry

        lax.fori_loop(0, nsuper, super_body, 0)
        pltpu.sync_copy(out_v, out.at[pl.ds(base, epw)])

    return sc_kernel


def kernel(x_user, x_movie, edge_label_index):
    n_user, dim = x_user.shape
    n_movie, _ = x_movie.shape
    e = edge_label_index.shape[1]
    eli = edge_label_index.astype(jnp.int32).reshape(-1)
    sc = _make_sc_kernel(n_user, n_movie, dim, e)
    return sc(x_user, x_movie, eli)


# speedup vs baseline: 2.3047x; 1.1011x over previous
import functools

import jax
import jax.numpy as jnp
from jax import lax
from jax.experimental import pallas as pl
from jax.experimental.pallas import tpu as pltpu
from jax.experimental.pallas import tpu_sc as plsc

L = 16
NC = 2
NS = 16
NW = NC * NS
CHUNK = 256
SUPER = 8192


def _make_sc_kernel(n_user, n_movie, dim, e):
    assert dim % L == 0 and e % (NW * SUPER) == 0
    epw = e // NW
    nch = SUPER // CHUNK
    nsuper = epw // SUPER
    mesh = plsc.VectorSubcoreMesh(core_axis_name="c", subcore_axis_name="s")

    @functools.partial(
        pl.kernel,
        mesh=mesh,
        compiler_params=pltpu.CompilerParams(
            needs_layout_passes=False, use_tc_tiling_on_sc=False),
        out_type=jax.ShapeDtypeStruct((e,), jnp.float32),
        scratch_types=[
            pltpu.VMEM((SUPER,), jnp.int32),
            pltpu.VMEM((SUPER,), jnp.int32),
            pltpu.VMEM((CHUNK, dim), jnp.float32),
            pltpu.VMEM((CHUNK, dim), jnp.float32),
            pltpu.VMEM((CHUNK, dim), jnp.float32),
            pltpu.VMEM((CHUNK, dim), jnp.float32),
            pltpu.VMEM((epw,), jnp.float32),
            pltpu.SemaphoreType.DMA,
            pltpu.SemaphoreType.DMA,
            pltpu.SemaphoreType.DMA,
            pltpu.SemaphoreType.DMA,
        ],
    )
    def sc_kernel(xu, xm, eli, out, uidx_v, midx_v, ru0, ru1, rm0, rm1,
                  out_v, su0, su1, sm0, sm1):
        cid = lax.axis_index("c")
        sid = lax.axis_index("s")
        wid = sid * NC + cid
        base = wid * epw
        iota16 = lax.iota(jnp.int32, L)
        rbufs = ((ru0, rm0, su0, sm0), (ru1, rm1, su1, sm1))

        def fire(b, j):
            ru, rm, su, sm = rbufs[b]
            jj = pl.multiple_of(j * CHUNK, CHUNK)
            pltpu.async_copy(xu.at[uidx_v.at[pl.ds(jj, CHUNK)]], ru, su)
            pltpu.async_copy(xm.at[midx_v.at[pl.ds(jj, CHUNK)]], rm, sm)

        def wait(b):
            ru, rm, su, sm = rbufs[b]
            pltpu.make_async_copy(xu.at[pl.ds(0, CHUNK)], ru, su).wait()
            pltpu.make_async_copy(xm.at[pl.ds(0, CHUNK)], rm, sm).wait()

        def compute_chunk(b, out_off):
            ru, rm, _, _ = rbufs[b]

            def group_body(g, carry):
                res = jnp.zeros((L,), jnp.float32)
                for el in range(L):
                    row = g * L + el
                    acc = ru[row, pl.ds(0, L)] * rm[row, pl.ds(0, L)]
                    for k in range(1, dim // L):
                        acc = acc + (ru[row, pl.ds(k * L, L)]
                                     * rm[row, pl.ds(k * L, L)])
                    res = jnp.where(iota16 == el, jnp.sum(acc), res)
                pred = 1.0 / (1.0 + jnp.exp(-res))
                out_v[pl.ds(out_off + g * L, L)] = pred
                return carry
            lax.fori_loop(0, CHUNK // L, group_body, 0)

        def super_body(s, carry):
            soff = pl.multiple_of(base + s * SUPER, SUPER)
            pltpu.sync_copy(eli.at[pl.ds(soff, SUPER)], uidx_v)
            pltpu.sync_copy(eli.at[pl.ds(soff + e, SUPER)], midx_v)

            fire(0, 0)

            def pair_body(jp, c2):
                j0 = jp * 2
                fire(1, j0 + 1)
                wait(0)
                compute_chunk(0, s * SUPER + j0 * CHUNK)

                @pl.when(j0 + 2 < nch)
                def _prefetch():
                    fire(0, j0 + 2)

                wait(1)
                compute_chunk(1, s * SUPER + (j0 + 1) * CHUNK)
                return c2
            lax.fori_loop(0, nch // 2, pair_body, 0)
            return carry

        lax.fori_loop(0, nsuper, super_body, 0)
        pltpu.sync_copy(out_v, out.at[pl.ds(base, epw)])

    return sc_kernel


def kernel(x_user, x_movie, edge_label_index):
    n_user, dim = x_user.shape
    n_movie, _ = x_movie.shape
    e = edge_label_index.shape[1]
    eli = edge_label_index.astype(jnp.int32).reshape(-1)
    sc = _make_sc_kernel(n_user, n_movie, dim, e)
    return sc(x_user, x_movie, eli)
